# baseline (speedup 1.0000x reference)
import functools

import jax
import jax.numpy as jnp
from jax import lax
from jax.experimental import pallas as pl
from jax.experimental.pallas import tpu as pltpu
from jax.experimental.pallas import tpu_sc as plsc

T, B, L, V, D = 26, 4096, 20, 100000, 16
NC, NS = 2, 16
NW = NC * NS
NB = B // NW
NBAGS = T * NB
LANES = 16
CHUNK = NB * L
LH = L // 2
HB = NB // 2
HROWS = LH * NB


def _emb_body(idx_hbm, len_hbm, tab_hbm, out_hbm,
              idxraw0, idxraw1, rows0, rows1, outb0, outb1, oidx0, oidx1,
              lens, gsem, osem, isem):
    sid = lax.axis_index("s")
    wid = sid * NC + lax.axis_index("c")
    b0 = wid * NB
    iota = lax.iota(jnp.int32, LANES)
    iotaT = iota * T
    iotaL = iota * L

    pltpu.sync_copy(len_hbm.at[:, pl.ds(b0, NB)], lens)

    idxbufs = (idxraw0, idxraw1)
    rowbufs = (rows0, rows1)
    outbufs = (outb0, outb1)
    oidxbufs = (oidx0, oidx1)
    tmax = T - 1

    def start_idx(t, raw):
        pltpu.async_copy(idx_hbm.at[pl.ds((t * B + b0) * L, CHUNK)], raw,
                         isem)

    def wait_idx(raw):
        pltpu.make_async_copy(idx_hbm.at[pl.ds(b0 * L, CHUNK)], raw,
                              isem).wait()

    def build(t, raw):
        toff = t * V

        def abody(g, carry):
            sl = pl.ds(g * LANES, LANES)
            raw[sl] = (raw[sl] & 0) + (g * LANES + toff) + iota
            return carry

        lax.fori_loop(0, CHUNK // LANES, abody, 0)

    def fire_gathers(raw, half, rows):
        for j in range(LH):
            pltpu.async_copy(
                tab_hbm.at[raw.at[pl.ds((half * LH + j) * NB, NB)]],
                rows.at[pl.ds(j * NB, NB)], gsem)

    def drain_gathers(raw, half, rows):
        for j in range(LH):
            pltpu.make_async_copy(
                tab_hbm.at[raw.at[pl.ds((half * LH + j) * NB, NB)]],
                rows.at[pl.ds(j * NB, NB)], gsem).wait()

    def accumulate(t, half, rows, outb, oidx):
        def cbody(c, carry):
            lensrow = lens[t, pl.ds(half * HB + c * LANES, LANES)]
            inv = 1.0 / jnp.maximum(lensrow, 1).astype(jnp.float32)
            accs = [jnp.zeros((LANES,), jnp.float32) for _ in range(D)]
            for l in range(L):
                rowsel = iotaL + (c * (LANES * L) + l)
                wvec = jnp.where(lensrow > l, inv, 0.0)
                for d in range(D):
                    dvec = jnp.full((LANES,), d, jnp.int32)
                    v = plsc.load_gather(rows, [rowsel, dvec])
                    accs[d] = accs[d] + v * wvec
            outsel = c * LANES + iota
            for d in range(D):
                dvec = jnp.full((LANES,), d, jnp.int32)
                plsc.store_scatter(outb, [outsel, dvec], accs[d])
            oidx[0, pl.ds(c * LANES, LANES)] = iotaT + (
                (b0 + half * HB + c * LANES) * T + t)
            return carry

        lax.fori_loop(0, HB // LANES, cbody, 0)

    def fire_out(outb, oidx):
        pltpu.async_copy(outb, out_hbm.at[oidx.at[0]], osem)

    def drain_out(outb, oidx):
        pltpu.make_async_copy(outb, out_hbm.at[oidx.at[0]], osem).wait()

    start_idx(0, idxraw0)
    wait_idx(idxraw0)
    build(0, idxraw0)
    start_idx(1, idxraw1)
    fire_gathers(idxraw0, 0, rows0)

    def sbody(s, carry):
        for q in range(4):
            t = 2 * s + (q // 2)
            iset = q // 2
            half = q % 2
            rawc = idxbufs[iset]
            rowsc = rowbufs[half]
            outc = outbufs[half]
            oidxc = oidxbufs[half]
            if half == 0:
                tn = jnp.minimum(t + 1, tmax)
                rawn = idxbufs[1 - iset]
                wait_idx(rawn)
                build(tn, rawn)
                gnext, hnext = rawc, 1
            else:
                gnext, hnext = idxbufs[1 - iset], 0
            drain_gathers(rawc, half, rowsc)
            if half == 1:
                start_idx(jnp.minimum(t + 2, tmax), rawc)
            fire_gathers(gnext, hnext, rowbufs[1 - half])
            if q >= 2:
                drain_out(outc, oidxc)
            else:
                @pl.when(s > 0)
                def _():
                    drain_out(outc, oidxc)
            accumulate(t, half, rowsc, outc, oidxc)
            fire_out(outc, oidxc)
        return carry

    lax.fori_loop(0, T // 2, sbody, 0)

    wait_idx(idxraw0)
    drain_gathers(idxraw0, 0, rows0)
    drain_out(outb0, oidx0)
    drain_out(outb1, oidx1)


_emb = functools.partial(
    pl.kernel,
    compiler_params=pltpu.CompilerParams(
        needs_layout_passes=False, use_tc_tiling_on_sc=False),
    out_type=jax.ShapeDtypeStruct((B * T, D), jnp.float32),
    mesh=plsc.VectorSubcoreMesh(core_axis_name="c", subcore_axis_name="s",
                                num_cores=NC, num_subcores=NS),
    scratch_types=[
        pltpu.VMEM((CHUNK,), jnp.int32),
        pltpu.VMEM((CHUNK,), jnp.int32),
        pltpu.VMEM((HROWS, D), jnp.float32),
        pltpu.VMEM((HROWS, D), jnp.float32),
        pltpu.VMEM((HB, D), jnp.float32),
        pltpu.VMEM((HB, D), jnp.float32),
        pltpu.VMEM((1, HB), jnp.int32),
        pltpu.VMEM((1, HB), jnp.int32),
        pltpu.VMEM((T, NB), jnp.int32),
        pltpu.SemaphoreType.DMA,
        pltpu.SemaphoreType.DMA,
        pltpu.SemaphoreType.DMA,
    ],
)(_emb_body)


def kernel(indices, lengths, tables):
    tab2d = tables.reshape(T * V, D)
    idxflat = indices.reshape(T * B * L)
    out2d = _emb(idxflat, lengths, tab2d)
    return out2d.reshape(B, T * D)


# speedup vs baseline: 1.2439x; 1.2439x over previous
import functools

import jax
import jax.numpy as jnp
from jax import lax
from jax.experimental import pallas as pl
from jax.experimental.pallas import tpu as pltpu
from jax.experimental.pallas import tpu_sc as plsc

T, B, L, V, D = 26, 4096, 20, 100000, 16
NC, NS = 2, 16
NW = NC * NS
NB = B // NW
NBAGS = T * NB
LANES = 16
CHUNK = NB * L
LH = L // 2
HB = NB // 2
HROWS = LH * NB


def _emb_body(idx_hbm, len_hbm, tab_hbm, out_hbm,
              idxraw0, idxraw1, rows0, rows1, outb0, outb1, oidx0, oidx1,
              lens, gsem, osem, isem):
    sid = lax.axis_index("s")
    wid = sid * NC + lax.axis_index("c")
    b0 = wid * NB
    iota = lax.iota(jnp.int32, LANES)
    iotaT = iota * T

    pltpu.sync_copy(len_hbm.at[:, pl.ds(b0, NB)], lens)

    idxbufs = (idxraw0, idxraw1)
    rowbufs = (rows0, rows1)
    outbufs = (outb0, outb1)
    oidxbufs = (oidx0, oidx1)
    tmax = T - 1

    def start_idx(t, raw):
        pltpu.async_copy(idx_hbm.at[pl.ds((t * B + b0) * L, CHUNK)], raw,
                         isem)

    def wait_idx(raw):
        pltpu.make_async_copy(idx_hbm.at[pl.ds(b0 * L, CHUNK)], raw,
                              isem).wait()

    def build(t, raw):
        toff = t * V

        def abody(g, carry):
            sl = pl.ds(g * LANES, LANES)
            raw[sl] = raw[sl] + toff
            return carry

        lax.fori_loop(0, CHUNK // LANES, abody, 0)

    def fire_gathers(raw, half, rows):
        pltpu.async_copy(tab_hbm.at[raw.at[pl.ds(half * HROWS, HROWS)]],
                         rows, gsem)

    def drain_gathers(raw, half, rows):
        pltpu.make_async_copy(tab_hbm.at[raw.at[pl.ds(half * HROWS, HROWS)]],
                              rows, gsem).wait()

    def accumulate(t, half, rows, outb, oidx):
        def bbody(c, carry):
            lensrow = lens[t, pl.ds(half * HB + c * LANES, LANES)]
            invrow = 1.0 / jnp.maximum(lensrow, 1).astype(jnp.float32)
            for i in range(LANES):
                len_s = lensrow[i]
                inv_s = invrow[i]
                base = (c * LANES + i) * L
                acc = jnp.zeros((LANES,), jnp.float32)
                for l in range(L):
                    w = jnp.where(len_s > l, inv_s, 0.0)
                    acc = acc + rows[base + l, :] * w
                outb[c * LANES + i, :] = acc
            return carry

        lax.fori_loop(0, HB // LANES, bbody, 0)

        def obody(c, carry):
            oidx[0, pl.ds(c * LANES, LANES)] = iotaT + (
                (b0 + half * HB + c * LANES) * T + t)
            return carry

        lax.fori_loop(0, HB // LANES, obody, 0)

    def fire_out(outb, oidx):
        pltpu.async_copy(outb, out_hbm.at[oidx.at[0]], osem)

    def drain_out(outb, oidx):
        pltpu.make_async_copy(outb, out_hbm.at[oidx.at[0]], osem).wait()

    start_idx(0, idxraw0)
    wait_idx(idxraw0)
    build(0, idxraw0)
    start_idx(1, idxraw1)
    fire_gathers(idxraw0, 0, rows0)

    def sbody(s, carry):
        for q in range(4):
            t = 2 * s + (q // 2)
            iset = q // 2
            half = q % 2
            rawc = idxbufs[iset]
            rowsc = rowbufs[half]
            outc = outbufs[half]
            oidxc = oidxbufs[half]
            if half == 0:
                tn = jnp.minimum(t + 1, tmax)
                rawn = idxbufs[1 - iset]
                wait_idx(rawn)
                build(tn, rawn)
                gnext, hnext = rawc, 1
            else:
                gnext, hnext = idxbufs[1 - iset], 0
            drain_gathers(rawc, half, rowsc)
            if half == 1:
                start_idx(jnp.minimum(t + 2, tmax), rawc)
            fire_gathers(gnext, hnext, rowbufs[1 - half])
            if q >= 2:
                drain_out(outc, oidxc)
            else:
                @pl.when(s > 0)
                def _():
                    drain_out(outc, oidxc)
            accumulate(t, half, rowsc, outc, oidxc)
            fire_out(outc, oidxc)
        return carry

    lax.fori_loop(0, T // 2, sbody, 0)

    wait_idx(idxraw0)
    drain_gathers(idxraw0, 0, rows0)
    drain_out(outb0, oidx0)
    drain_out(outb1, oidx1)


_emb = functools.partial(
    pl.kernel,
    compiler_params=pltpu.CompilerParams(
        needs_layout_passes=False, use_tc_tiling_on_sc=False),
    out_type=jax.ShapeDtypeStruct((B * T, D), jnp.float32),
    mesh=plsc.VectorSubcoreMesh(core_axis_name="c", subcore_axis_name="s",
                                num_cores=NC, num_subcores=NS),
    scratch_types=[
        pltpu.VMEM((CHUNK,), jnp.int32),
        pltpu.VMEM((CHUNK,), jnp.int32),
        pltpu.VMEM((HROWS, D), jnp.float32),
        pltpu.VMEM((HROWS, D), jnp.float32),
        pltpu.VMEM((HB, D), jnp.float32),
        pltpu.VMEM((HB, D), jnp.float32),
        pltpu.VMEM((1, HB), jnp.int32),
        pltpu.VMEM((1, HB), jnp.int32),
        pltpu.VMEM((T, NB), jnp.int32),
        pltpu.SemaphoreType.DMA,
        pltpu.SemaphoreType.DMA,
        pltpu.SemaphoreType.DMA,
    ],
)(_emb_body)


def kernel(indices, lengths, tables):
    tab2d = tables.reshape(T * V, D)
    idxflat = indices.reshape(T * B * L)
    out2d = _emb(idxflat, lengths, tab2d)
    return out2d.reshape(B, T * D)
